# baseline (speedup 1.0000x reference)
import functools

import jax
import jax.numpy as jnp
from jax import lax
from jax.experimental import pallas as pl
from jax.experimental.pallas import tpu as pltpu
from jax.experimental.pallas import tpu_sc as plsc

NUM_EMB = 1_000_000
DIM = 16
BATCH = 16384
HIST = 50
N = BATCH * HIST

NC = 2
NS = 16
NW = NC * NS
N_PER_W = N // NW

SUB = 128
CHUNK = 1280
NSUB = CHUNK // SUB
NGROUP = CHUNK // 16
N_CHUNKS = N_PER_W // CHUNK


def _sc_body(x_hbm, y_hbm, w_hbm, out_hbm,
             xi0, xi1, yi0, yi1, xr0, xr1, yr0, yr1, o_v, sem):
    wid = lax.axis_index("s") * NC + lax.axis_index("c")
    row0_w = wid * (N_PER_W // SUB)
    xi = (xi0, xi1)
    yi = (yi0, yi1)
    xr = (xr0, xr1)
    yr = (yr0, yr1)

    def stage_idx(c, par):
        row0 = row0_w + c * NSUB
        pltpu.sync_copy(x_hbm.at[pl.ds(row0, NSUB)], xi[par])
        pltpu.sync_copy(y_hbm.at[pl.ds(row0, NSUB)], yi[par])

    def fire(par):
        for j in range(NSUB):
            pltpu.async_copy(
                w_hbm.at[xi[par].at[j]], xr[par].at[pl.ds(j * SUB, SUB)], sem)
            pltpu.async_copy(
                w_hbm.at[yi[par].at[j]], yr[par].at[pl.ds(j * SUB, SUB)], sem)

    def drain(par):
        pltpu.make_async_copy(w_hbm.at[pl.ds(0, CHUNK)], xr[par], sem).wait()
        pltpu.make_async_copy(w_hbm.at[pl.ds(0, CHUNK)], yr[par], sem).wait()

    def compute(c, par):
        base = (row0_w + c * NSUB) * SUB

        @plsc.parallel_loop(0, NGROUP, 1, unroll=4)
        def group_body(g):
            rows = g * 16 + lax.iota(jnp.int32, 16)
            acc = None
            for i in range(DIM):
                col = jnp.full((16,), i, jnp.int32)
                di = jnp.abs(plsc.load_gather(xr[par], [rows, col])
                             - plsc.load_gather(yr[par], [rows, col]))
                acc = di if acc is None else acc + di
            plsc.store_scatter(o_v, [rows], acc)
        pltpu.sync_copy(o_v, out_hbm.at[pl.ds(base, CHUNK)])

    stage_idx(0, 0)
    fire(0)

    def pair_body(p, carry):
        for par in (0, 1):
            c = 2 * p + par
            drain(par)

            @pl.when(c + 1 < N_CHUNKS)
            def _():
                stage_idx(c + 1, 1 - par)
                fire(1 - par)

            compute(c, par)
        return carry

    lax.fori_loop(0, N_CHUNKS // 2, pair_body, 0, unroll=False)


GCOL = 4
GIDS = GCOL * 128
GNG = GIDS // 16
COLS_MAIN = 7808
COLS_PER_W = COLS_MAIN // NW
GNCH = COLS_PER_W // GCOL
TAIL0 = COLS_MAIN * 128
LAST_COL_IDS = NUM_EMB - 7812 * 128


def _g16(v):
    m = v[0]
    for u in v[1:]:
        m = jnp.maximum(m, u)
    e = [jnp.exp(u - m) for u in v]
    s = e[0]
    for u in e[1:]:
        s = s + u
    t4 = e[4] + e[13] + e[14] + e[15]
    t3 = e[3] + e[10] + e[11] + e[12]
    t2 = e[2] + e[7] + e[8] + e[9]
    t1 = e[1] + e[5] + e[6] + t4
    t0 = e[0] + t1 + t2 + t3
    inv = 1.0 / s
    return [t * inv for t in
            [t0, t1, t2, t3, t4] + e[5:]]


def _ga_body(wt_hbm, wtail_hbm, g_hbm, in0, in1, o0, o1, tail_v, tcol_v,
             semi, semo):
    wid = lax.axis_index("s") * NC + lax.axis_index("c")
    col_w = wid * COLS_PER_W
    inb = (in0, in1)
    ob = (o0, o1)
    lane16 = lax.iota(jnp.int32, 16) * 16

    def fire_in(ch, par):
        pltpu.async_copy(
            wt_hbm.at[pl.ds(0, DIM), pl.ds((col_w + ch * GCOL) * 128, GIDS)],
            inb[par], semi)

    def drain_in(par):
        pltpu.make_async_copy(
            wt_hbm.at[pl.ds(0, DIM), pl.ds(0, GIDS)], inb[par], semi).wait()

    def drain_out(par):
        pltpu.make_async_copy(
            g_hbm.at[pl.ds(0, GIDS * DIM)], ob[par], semo).wait()

    def compute_groups(src, dst, n_groups):
        @plsc.parallel_loop(0, n_groups, 1, unroll=4)
        def group_body(g):
            l0 = g * 16
            v = [src[d, pl.ds(l0, 16)] for d in range(DIM)]
            gv = _g16(v)
            addr0 = l0 * 16 + lane16
            for j in range(DIM):
                plsc.store_scatter(dst, [addr0 + j], gv[j])

    fire_in(0, 0)

    def pair_body(p, carry):
        for par in (0, 1):
            ch = 2 * p + par

            @pl.when(ch < GNCH)
            def _():
                drain_in(par)

                @pl.when(ch + 1 < GNCH)
                def _():
                    fire_in(ch + 1, 1 - par)

                @pl.when(ch >= 2)
                def _():
                    drain_out(par)

                compute_groups(inb[par], ob[par], GNG)
                pltpu.async_copy(
                    ob[par],
                    g_hbm.at[pl.ds((col_w + ch * GCOL) * 2048, GIDS * DIM)],
                    semo)
        return carry

    lax.fori_loop(0, (GNCH + 1) // 2, pair_body, 0, unroll=False)
    drain_out(0)
    drain_out(1)

    @pl.when(wid < 4)
    def _():
        col = COLS_MAIN + wid
        pltpu.sync_copy(
            wt_hbm.at[pl.ds(0, DIM), pl.ds(col * 128, 128)], tcol_v)
        compute_groups(tcol_v, o0, 8)
        pltpu.sync_copy(o0.at[pl.ds(0, 2048)],
                        g_hbm.at[pl.ds(col * 2048, 2048)])

    @pl.when(wid == 4)
    def _():
        pltpu.sync_copy(wtail_hbm, tail_v)
        compute_groups(tail_v, o0, LAST_COL_IDS // 16)
        pltpu.sync_copy(o0.at[pl.ds(0, LAST_COL_IDS * DIM)],
                        g_hbm.at[pl.ds(7812 * 2048, LAST_COL_IDS * DIM)])


@jax.jit
def _build_gtable(wt, wtail):
    mesh = plsc.VectorSubcoreMesh(core_axis_name="c", subcore_axis_name="s")
    f = pl.kernel(
        _ga_body,
        mesh=mesh,
        out_type=jax.ShapeDtypeStruct((NUM_EMB * DIM,), jnp.float32),
        scratch_types=[
            pltpu.VMEM((DIM, GIDS), jnp.float32),
            pltpu.VMEM((DIM, GIDS), jnp.float32),
            pltpu.VMEM((GIDS * DIM,), jnp.float32),
            pltpu.VMEM((GIDS * DIM,), jnp.float32),
            pltpu.VMEM((DIM, LAST_COL_IDS), jnp.float32),
            pltpu.VMEM((DIM, 128), jnp.float32),
            pltpu.SemaphoreType.DMA,
            pltpu.SemaphoreType.DMA,
        ],
        compiler_params=pltpu.CompilerParams(
            needs_layout_passes=False, use_tc_tiling_on_sc=True),
    )
    return f(wt, wtail)


@jax.jit
def _poincare_sc(x2d, y2d, weight):
    mesh = plsc.VectorSubcoreMesh(core_axis_name="c", subcore_axis_name="s")
    f = pl.kernel(
        _sc_body,
        mesh=mesh,
        out_type=jax.ShapeDtypeStruct((N,), jnp.float32),
        scratch_types=[
            pltpu.VMEM((NSUB, SUB), jnp.int32),
            pltpu.VMEM((NSUB, SUB), jnp.int32),
            pltpu.VMEM((NSUB, SUB), jnp.int32),
            pltpu.VMEM((NSUB, SUB), jnp.int32),
            pltpu.VMEM((CHUNK, DIM), jnp.float32),
            pltpu.VMEM((CHUNK, DIM), jnp.float32),
            pltpu.VMEM((CHUNK, DIM), jnp.float32),
            pltpu.VMEM((CHUNK, DIM), jnp.float32),
            pltpu.VMEM((CHUNK,), jnp.float32),
            pltpu.SemaphoreType.DMA,
        ],
        compiler_params=pltpu.CompilerParams(
            needs_layout_passes=False, use_tc_tiling_on_sc=False),
    )
    return f(x2d, y2d, weight)


def kernel(x, y, weight):
    x2d = x.T.reshape(N // SUB, SUB)
    y2d = y.T.reshape(N // SUB, SUB)
    wt = weight.T
    w_lin = _build_gtable(wt, wt[:, 7812 * 128:]).reshape(NUM_EMB, DIM)
    out = _poincare_sc(x2d, y2d, w_lin)
    return out.reshape(HIST, BATCH).T


# speedup vs baseline: 1.4549x; 1.4549x over previous
import functools

import jax
import jax.numpy as jnp
from jax import lax
from jax.experimental import pallas as pl
from jax.experimental.pallas import tpu as pltpu
from jax.experimental.pallas import tpu_sc as plsc

NUM_EMB = 1_000_000
DIM = 16
BATCH = 16384
HIST = 50
N = BATCH * HIST

NC = 2
NS = 16
NW = NC * NS
N_PER_W = N // NW

SUB = 128
CHUNK = 1280
NSUB = CHUNK // SUB
NGROUP = CHUNK // 16
N_CHUNKS = N_PER_W // CHUNK


def _sc_body(x_hbm, y_hbm, w_hbm, out_hbm,
             xi0, xi1, yi0, yi1, xr0, xr1, yr0, yr1, o_v, sem, semi):
    wid = lax.axis_index("s") * NC + lax.axis_index("c")
    row0_w = wid * (N_PER_W // SUB)
    xi = (xi0, xi1)
    yi = (yi0, yi1)
    xr = (xr0, xr1)
    yr = (yr0, yr1)

    def stage_idx(c, par):
        row0 = row0_w + c * NSUB
        pltpu.async_copy(x_hbm.at[pl.ds(row0, NSUB)], xi[par], semi)
        pltpu.async_copy(y_hbm.at[pl.ds(row0, NSUB)], yi[par], semi)

    def drain_idx(par):
        pltpu.make_async_copy(x_hbm.at[pl.ds(0, NSUB)], xi[par], semi).wait()
        pltpu.make_async_copy(y_hbm.at[pl.ds(0, NSUB)], yi[par], semi).wait()

    def fire(par):
        for j in range(NSUB):
            pltpu.async_copy(
                w_hbm.at[xi[par].at[j]], xr[par].at[pl.ds(j * SUB, SUB)], sem)
            pltpu.async_copy(
                w_hbm.at[yi[par].at[j]], yr[par].at[pl.ds(j * SUB, SUB)], sem)

    def drain(par):
        pltpu.make_async_copy(w_hbm.at[pl.ds(0, CHUNK)], xr[par], sem).wait()
        pltpu.make_async_copy(w_hbm.at[pl.ds(0, CHUNK)], yr[par], sem).wait()

    def compute(c, par):
        base = (row0_w + c * NSUB) * SUB

        @plsc.parallel_loop(0, NGROUP, 1, unroll=2)
        def group_body(g):
            rows = g * 16 + lax.iota(jnp.int32, 16)
            acc = None
            for i in range(1, DIM):
                col = jnp.full((16,), i, jnp.int32)
                di = jnp.abs(plsc.load_gather(xr[par], [rows, col])
                             - plsc.load_gather(yr[par], [rows, col]))
                acc = di if acc is None else acc + di
            plsc.store_scatter(o_v, [rows], acc)
        pltpu.sync_copy(o_v, out_hbm.at[pl.ds(base, CHUNK)])

    stage_idx(0, 0)
    drain_idx(0)
    fire(0)
    stage_idx(1, 1)

    def pair_body(p, carry):
        for par in (0, 1):
            c = 2 * p + par
            drain(par)

            @pl.when(c + 1 < N_CHUNKS)
            def _():
                drain_idx(1 - par)
                fire(1 - par)

                @pl.when(c + 2 < N_CHUNKS)
                def _():
                    stage_idx(c + 2, par)

            compute(c, par)
        return carry

    lax.fori_loop(0, N_CHUNKS // 2, pair_body, 0, unroll=False)


GCOL = 4
GIDS = GCOL * 128
GNG = GIDS // 16
COLS_MAIN = 7808
COLS_PER_W = COLS_MAIN // NW
GNCH = COLS_PER_W // GCOL
TAIL0 = COLS_MAIN * 128
LAST_COL_IDS = NUM_EMB - 7812 * 128


def _g16(v):
    m = v[0]
    for u in v[1:]:
        m = jnp.maximum(m, u)
    e = [jnp.exp(u - m) for u in v]
    s = e[0]
    for u in e[1:]:
        s = s + u
    t4 = e[4] + e[13] + e[14] + e[15]
    t3 = e[3] + e[10] + e[11] + e[12]
    t2 = e[2] + e[7] + e[8] + e[9]
    t1 = e[1] + e[5] + e[6] + t4
    t0 = e[0] + t1 + t2 + t3
    inv = 1.0 / s
    return [t * inv for t in
            [t0, t1, t2, t3, t4] + e[5:]]


def _ga_body(wt_hbm, wtail_hbm, g_hbm, in0, in1, o0, o1, tail_v, tcol_v,
             semi, semo):
    wid = lax.axis_index("s") * NC + lax.axis_index("c")
    col_w = wid * COLS_PER_W
    inb = (in0, in1)
    ob = (o0, o1)
    lane16 = lax.iota(jnp.int32, 16) * 16

    def fire_in(ch, par):
        pltpu.async_copy(
            wt_hbm.at[pl.ds(0, DIM), pl.ds((col_w + ch * GCOL) * 128, GIDS)],
            inb[par], semi)

    def drain_in(par):
        pltpu.make_async_copy(
            wt_hbm.at[pl.ds(0, DIM), pl.ds(0, GIDS)], inb[par], semi).wait()

    def drain_out(par):
        pltpu.make_async_copy(
            g_hbm.at[pl.ds(0, GIDS * DIM)], ob[par], semo).wait()

    def compute_groups(src, dst, n_groups):
        @plsc.parallel_loop(0, n_groups, 1, unroll=2)
        def group_body(g):
            l0 = g * 16
            v = [src[d, pl.ds(l0, 16)] for d in range(DIM)]
            gv = _g16(v)
            addr0 = l0 * 16 + lane16
            for j in range(DIM):
                plsc.store_scatter(dst, [addr0 + j], gv[j])

    fire_in(0, 0)

    def pair_body(p, carry):
        for par in (0, 1):
            ch = 2 * p + par

            @pl.when(ch < GNCH)
            def _():
                drain_in(par)

                @pl.when(ch + 1 < GNCH)
                def _():
                    fire_in(ch + 1, 1 - par)

                @pl.when(ch >= 2)
                def _():
                    drain_out(par)

                compute_groups(inb[par], ob[par], GNG)
                pltpu.async_copy(
                    ob[par],
                    g_hbm.at[pl.ds((col_w + ch * GCOL) * 2048, GIDS * DIM)],
                    semo)
        return carry

    lax.fori_loop(0, (GNCH + 1) // 2, pair_body, 0, unroll=False)
    drain_out(0)
    drain_out(1)

    @pl.when(wid < 4)
    def _():
        col = COLS_MAIN + wid
        pltpu.sync_copy(
            wt_hbm.at[pl.ds(0, DIM), pl.ds(col * 128, 128)], tcol_v)
        compute_groups(tcol_v, o0, 8)
        pltpu.sync_copy(o0.at[pl.ds(0, 2048)],
                        g_hbm.at[pl.ds(col * 2048, 2048)])

    @pl.when(wid == 4)
    def _():
        pltpu.sync_copy(wtail_hbm, tail_v)
        compute_groups(tail_v, o0, LAST_COL_IDS // 16)
        pltpu.sync_copy(o0.at[pl.ds(0, LAST_COL_IDS * DIM)],
                        g_hbm.at[pl.ds(7812 * 2048, LAST_COL_IDS * DIM)])


@jax.jit
def _build_gtable(wt, wtail):
    mesh = plsc.VectorSubcoreMesh(core_axis_name="c", subcore_axis_name="s")
    f = pl.kernel(
        _ga_body,
        mesh=mesh,
        out_type=jax.ShapeDtypeStruct((NUM_EMB * DIM,), jnp.float32),
        scratch_types=[
            pltpu.VMEM((DIM, GIDS), jnp.float32),
            pltpu.VMEM((DIM, GIDS), jnp.float32),
            pltpu.VMEM((GIDS * DIM,), jnp.float32),
            pltpu.VMEM((GIDS * DIM,), jnp.float32),
            pltpu.VMEM((DIM, LAST_COL_IDS), jnp.float32),
            pltpu.VMEM((DIM, 128), jnp.float32),
            pltpu.SemaphoreType.DMA,
            pltpu.SemaphoreType.DMA,
        ],
        compiler_params=pltpu.CompilerParams(
            needs_layout_passes=False, use_tc_tiling_on_sc=True),
    )
    return f(wt, wtail)


@jax.jit
def _poincare_sc(x2d, y2d, weight):
    mesh = plsc.VectorSubcoreMesh(core_axis_name="c", subcore_axis_name="s")
    f = pl.kernel(
        _sc_body,
        mesh=mesh,
        out_type=jax.ShapeDtypeStruct((N,), jnp.float32),
        scratch_types=[
            pltpu.VMEM((NSUB, SUB), jnp.int32),
            pltpu.VMEM((NSUB, SUB), jnp.int32),
            pltpu.VMEM((NSUB, SUB), jnp.int32),
            pltpu.VMEM((NSUB, SUB), jnp.int32),
            pltpu.VMEM((CHUNK, DIM), jnp.float32),
            pltpu.VMEM((CHUNK, DIM), jnp.float32),
            pltpu.VMEM((CHUNK, DIM), jnp.float32),
            pltpu.VMEM((CHUNK, DIM), jnp.float32),
            pltpu.VMEM((CHUNK,), jnp.float32),
            pltpu.SemaphoreType.DMA,
            pltpu.SemaphoreType.DMA,
        ],
        compiler_params=pltpu.CompilerParams(
            needs_layout_passes=False, use_tc_tiling_on_sc=False),
    )
    return f(x2d, y2d, weight)


def kernel(x, y, weight):
    x2d = x.T.reshape(N // SUB, SUB)
    y2d = y.T.reshape(N // SUB, SUB)
    wt = weight.T
    w_lin = _build_gtable(wt, wt[:, 7812 * 128:]).reshape(NUM_EMB, DIM)
    out = _poincare_sc(x2d, y2d, w_lin)
    return out.reshape(HIST, BATCH).T
